# baseline (speedup 1.0000x reference)
import functools

import jax
import jax.numpy as jnp
from jax import lax
from jax.experimental import pallas as pl
from jax.experimental.pallas import tpu as pltpu
from jax.experimental.pallas import tpu_sc as plsc

NW = 32
SB = 128
SBS = 80
BE = 640
BN = 512
NCHUNK = 4

_INV_SQRT2 = 0.7071067811865476


def _gelu(z):
    return 0.5 * z * (1.0 + lax.erf(z * _INV_SQRT2))


def _ln_rows(xb, g, b, eps=1e-5):
    m = jnp.mean(xb, axis=1, keepdims=True)
    v = jnp.mean((xb - m) ** 2, axis=1, keepdims=True)
    return (xb - m) / jnp.sqrt(v + eps) * g + b


def _pack2(a, b):
    au = lax.bitcast_convert_type(a.astype(jnp.bfloat16).astype(jnp.float32),
                                  jnp.int32)
    bu = lax.bitcast_convert_type(b.astype(jnp.bfloat16).astype(jnp.float32),
                                  jnp.int32)
    return lax.bitcast_convert_type(au | lax.shift_right_logical(bu, 16),
                                    jnp.float32)


def _unpack_hi(p):
    u = lax.bitcast_convert_type(p, jnp.int32)
    return lax.bitcast_convert_type(u & jnp.int32(-65536), jnp.float32)


def _unpack_lo(p):
    u = lax.bitcast_convert_type(p, jnp.int32)
    return lax.bitcast_convert_type(lax.shift_left(u, 16), jnp.float32)


def _prep_body(x_ref, g_ref, b_ref, W_ref, bc_ref, h_ref, ts_ref, td_ref):
    h = _ln_rows(x_ref[...], g_ref[...], b_ref[...])
    h_ref[...] = h
    P = jnp.dot(h, W_ref[...], preferred_element_type=jnp.float32) + bc_ref[...]
    ts_ref[...] = _pack2(P[:, :128], P[:, 128:256])
    td_ref[...] = P[:, 256:]


def _mlp_body(gs_ref, gd_ref, v_ref, emb_ref, Wem_ref, Weg_ref, Wm2_ref,
              bm2_ref, wg2_ref, bg2_ref, id_ref, out_ref):
    gs = gs_ref[...]
    gd = gd_ref[...]
    e = emb_ref[...]
    em = jnp.dot(e, Wem_ref[...], preferred_element_type=jnp.float32)
    eg = jnp.dot(e, Weg_ref[...], preferred_element_type=jnp.float32)
    mpre = _unpack_hi(gs) + em
    gpre = gd + _unpack_lo(gs) + eg
    msg = jnp.dot(_gelu(mpre), Wm2_ref[...],
                  preferred_element_type=jnp.float32) + bm2_ref[...]
    glog = jnp.sum(_gelu(gpre) * wg2_ref[...], axis=1, keepdims=True) + bg2_ref[0, 0]
    mg = msg * jax.nn.sigmoid(glog)
    ident = id_ref[...]
    vb = v_ref[...].reshape(BE // 128, 128)
    for c in range(BE // 128):
        vcol = jnp.sum(ident * vb[c:c + 1, :], axis=1, keepdims=True)
        out_ref[c * 128:(c + 1) * 128, :] = mg[c * 128:(c + 1) * 128, :] * vcol


def kernel(x, edge_src, edge_dst, edge_emb, time_weight, message_node_scale,
           ln1_g, ln1_b, ln2_g, ln2_b, W_self, b_self, W_msg1, b_msg1,
           W_msg2, b_msg2, W_gate1, b_gate1, W_gate2, b_gate2, W_agg, b_agg,
           W_ffn1, b_ffn1, W_ffn2, b_ffn2):
    N, D = x.shape
    E = edge_src.shape[0]
    f32 = jnp.float32

    NP = -(-N // 2048) * 2048
    n_stripe = NP // 16 // SBS

    units = E // 2560
    w = [3, 5, 6, 2][:NCHUNK] if NCHUNK == 4 else [1] * NCHUNK
    tot = sum(w)
    uk = [units * wi // tot for wi in w]
    uk[-1] += units - sum(uk)
    cuts = [0]
    for k in range(NCHUNK):
        cuts.append(cuts[-1] + 2560 * uk[k])

    xp = jnp.pad(x, ((0, NP - N), (0, 0)))
    src_p = edge_src.astype(jnp.int32)
    dst_p = edge_dst.astype(jnp.int32)
    tw_f = time_weight.reshape(E)
    s_f = jnp.pad(message_node_scale, ((0, NP - N), (0, 0))).reshape(NP)

    Wcat = jnp.concatenate([W_msg1[:D], W_gate1[D:2 * D], W_gate1[:D]], axis=1)
    bcat = jnp.concatenate([b_msg1, jnp.zeros((D,), f32), b_gate1])[None, :]
    Wem = W_msg1[D:]
    Weg = W_gate1[2 * D:]
    ident = jnp.eye(128, dtype=f32)

    row1 = lambda a: a[None, :]

    h, t_src, t_dst = pl.pallas_call(
        _prep_body,
        grid=(NP // BN,),
        in_specs=[
            pl.BlockSpec((BN, D), lambda i: (i, 0)),
            pl.BlockSpec((1, D), lambda i: (0, 0)),
            pl.BlockSpec((1, D), lambda i: (0, 0)),
            pl.BlockSpec((D, 3 * D), lambda i: (0, 0)),
            pl.BlockSpec((1, 3 * D), lambda i: (0, 0)),
        ],
        out_specs=[
            pl.BlockSpec((BN, D), lambda i: (i, 0)),
            pl.BlockSpec((BN, D), lambda i: (i, 0)),
            pl.BlockSpec((BN, D), lambda i: (i, 0)),
        ],
        out_shape=[
            jax.ShapeDtypeStruct((NP, D), f32),
            jax.ShapeDtypeStruct((NP, D), f32),
            jax.ShapeDtypeStruct((NP, D), f32),
        ],
    )(xp, row1(ln1_g), row1(ln1_b), Wcat, bcat)

    mesh = plsc.VectorSubcoreMesh(core_axis_name="c", subcore_axis_name="s")
    sc_params = pltpu.CompilerParams(use_tc_tiling_on_sc=False,
                                     needs_layout_passes=False)

    def make_gather(e0, ne):
        b0 = e0 // SB
        nb = ne // SB
        nb_lo = nb // NW
        rem = nb - nb_lo * NW

        @functools.partial(
            pl.kernel,
            out_type=[
                jax.ShapeDtypeStruct((ne, D), f32),
                jax.ShapeDtypeStruct((ne, D), f32),
                jax.ShapeDtypeStruct((nb, SB), f32),
            ],
            mesh=mesh,
            scratch_types=[
                pltpu.VMEM((SB,), jnp.int32),
                pltpu.VMEM((SB,), jnp.int32),
                pltpu.VMEM((SB,), f32),
                pltpu.VMEM((SB,), f32),
                pltpu.VMEM((NP,), f32),
                pltpu.VMEM((SB, D), f32),
                pltpu.VMEM((SB, D), f32),
                pltpu.SemaphoreType.DMA,
                pltpu.SemaphoreType.DMA,
            ],
            compiler_params=sc_params,
        )
        def _gather(ts_hbm, td_hbm, src_hbm, dst_hbm, tw_hbm, s_hbm,
                    gsrc_hbm, gdst_hbm, v_hbm,
                    isrc, idst, twb, vb, s_v, bufA, bufB, semA, semB):
            wid = lax.axis_index("s") * 2 + lax.axis_index("c")
            base_blk = nb_lo * wid + jnp.minimum(wid, rem)
            pltpu.sync_copy(s_hbm, s_v)

            def body(g, carry):
                lblk = jnp.minimum(base_blk + g, nb - 1)
                base = e0 + lblk * SB
                lbase = lblk * SB
                pltpu.sync_copy(src_hbm.at[pl.ds(base, SB)], isrc)
                pltpu.sync_copy(dst_hbm.at[pl.ds(base, SB)], idst)
                pltpu.sync_copy(tw_hbm.at[pl.ds(base, SB)], twb)
                cpA = pltpu.async_copy(ts_hbm.at[isrc], bufA, semA)
                cpB = pltpu.async_copy(td_hbm.at[idst], bufB, semB)
                for i in range(SB // 16):
                    sl = pl.ds(i * 16, 16)
                    ss = plsc.load_gather(s_v, [isrc[sl]])
                    sd = plsc.load_gather(s_v, [idst[sl]])
                    vb[sl] = 0.5 * twb[sl] * (ss + sd)
                cpA.wait()
                cpB.wait()
                pltpu.sync_copy(bufA, gsrc_hbm.at[pl.ds(lbase, SB)])
                pltpu.sync_copy(bufB, gdst_hbm.at[pl.ds(lbase, SB)])
                pltpu.sync_copy(vb, v_hbm.at[lblk])
                return carry

            lax.fori_loop(0, nb_lo + (1 if rem else 0), body, 0)

        return _gather

    def make_scatter(e0, ne):
        n_sc_blk = ne // (NW * SBS)

        @functools.partial(
            pl.kernel,
            out_type=[
                jax.ShapeDtypeStruct((2, NP, D), f32),
                jax.ShapeDtypeStruct((NW, NP), f32),
            ],
            mesh=mesh,
            scratch_types=[
                pltpu.VMEM((SBS, D), f32),
                pltpu.VMEM((SBS,), jnp.int32),
                pltpu.VMEM((NP,), f32),
                pltpu.VMEM_SHARED((NP, D), f32),
            ],
            compiler_params=sc_params,
        )
        def _scatter(er_hbm, dst_hbm, z_hbm, agg_hbm, deg_hbm, rows, dstb,
                     deg_v, agg_sh):
            c = lax.axis_index("c")
            s = lax.axis_index("s")
            wid = s * 2 + c
            per_w = ne // NW
            zero16 = jnp.zeros((16,), f32)
            ones16 = jnp.ones((16,), f32)

            def zdeg(i, carry):
                deg_v[pl.ds(i * 16, 16)] = zero16
                return carry
            lax.fori_loop(0, NP // 16, zdeg, 0)

            pltpu.sync_copy(z_hbm, rows)
            for j in range(n_stripe):
                pltpu.sync_copy(rows,
                                agg_sh.at[pl.ds(s * (NP // 16) + j * SBS, SBS)])
            plsc.subcore_barrier()

            def body(g, carry):
                lbase = wid * per_w + g * SBS
                pltpu.sync_copy(dst_hbm.at[pl.ds(e0 + lbase, SBS)], dstb)
                pltpu.sync_copy(er_hbm.at[pl.ds(lbase, SBS)], rows)
                pltpu.sync_copy(rows, agg_sh.at[dstb], add=True)
                for i in range(SBS // 16):
                    sl = pl.ds(i * 16, 16)
                    plsc.addupdate_scatter(deg_v, [dstb[sl]], ones16)
                return carry

            lax.fori_loop(0, n_sc_blk, body, 0)
            plsc.subcore_barrier()

            for j in range(n_stripe):
                r0 = s * (NP // 16) + j * SBS
                pltpu.sync_copy(agg_sh.at[pl.ds(r0, SBS)], rows)
                pltpu.sync_copy(rows, agg_hbm.at[c].at[pl.ds(r0, SBS)])
            pltpu.sync_copy(deg_v, deg_hbm.at[wid])

        return _scatter

    zrows = jnp.zeros((SBS, D), f32)
    er_chunks, agg_parts, deg_parts = [], [], []
    for k in range(NCHUNK):
        e0, e1 = cuts[k], cuts[k + 1]
        ne = e1 - e0
        g_src, g_dst, v_blk = make_gather(e0, ne)(
            t_src, t_dst, src_p, dst_p, tw_f, s_f)
        v3 = v_blk.reshape(ne // SB, 1, SB)
        DE = edge_emb.shape[1]
        emb_k = edge_emb[e0:e1]
        er_k = pl.pallas_call(
            _mlp_body,
            grid=(ne // BE,),
            in_specs=[
                pl.BlockSpec((BE, D), lambda i: (i, 0)),
                pl.BlockSpec((BE, D), lambda i: (i, 0)),
                pl.BlockSpec((BE // SB, 1, SB), lambda i: (i, 0, 0)),
                pl.BlockSpec((BE, DE), lambda i: (i, 0)),
                pl.BlockSpec((DE, D), lambda i: (0, 0)),
                pl.BlockSpec((DE, D), lambda i: (0, 0)),
                pl.BlockSpec((D, D), lambda i: (0, 0)),
                pl.BlockSpec((1, D), lambda i: (0, 0)),
                pl.BlockSpec((1, D), lambda i: (0, 0)),
                pl.BlockSpec((1, 1), lambda i: (0, 0)),
                pl.BlockSpec((128, 128), lambda i: (0, 0)),
            ],
            out_specs=pl.BlockSpec((BE, D), lambda i: (i, 0)),
            out_shape=jax.ShapeDtypeStruct((ne, D), f32),
        )(g_src, g_dst, v3, emb_k, Wem, Weg,
          W_msg2, row1(b_msg2), W_gate2.reshape(1, D), b_gate2.reshape(1, 1),
          ident)
        er_chunks.append(er_k)
        ap_k, dp_k = make_scatter(e0, ne)(er_k, dst_p, zrows)
        agg_parts.append(ap_k)
        deg_parts.append(dp_k)

    edge_repr = (er_chunks[0] if NCHUNK == 1
                 else jnp.concatenate(er_chunks, axis=0))

    def _final_chunked(*refs):
        x_ref, h_ref = refs[0], refs[1]
        aps = refs[2:2 + NCHUNK]
        dps = refs[2 + NCHUNK:2 + 2 * NCHUNK]
        (Ws_ref, bs_ref, Wa_ref, ba_ref, g2_ref, b2_ref,
         Wf1_ref, bf1_ref, Wf2_ref, bf2_ref, out_ref) = refs[2 + 2 * NCHUNK:]
        deg = sum(jnp.sum(d[...], axis=0) for d in dps)
        agg_s = sum(a[0] + a[1] for a in aps)
        agg = agg_s / jnp.clip(deg, 1.0, None)[:, None]
        upd = (jnp.dot(h_ref[...], Ws_ref[...],
                       preferred_element_type=jnp.float32) + bs_ref[...]
               + jnp.dot(agg, Wa_ref[...],
                         preferred_element_type=jnp.float32) + ba_ref[...])
        o1 = x_ref[...] + upd
        f = _ln_rows(o1, g2_ref[...], b2_ref[...])
        ffn = jnp.dot(_gelu(jnp.dot(f, Wf1_ref[...],
                                    preferred_element_type=jnp.float32)
                            + bf1_ref[...]),
                      Wf2_ref[...], preferred_element_type=jnp.float32) + bf2_ref[...]
        out_ref[...] = o1 + ffn

    out = pl.pallas_call(
        _final_chunked,
        grid=(NP // BN,),
        in_specs=(
            [pl.BlockSpec((BN, D), lambda i: (i, 0)),
             pl.BlockSpec((BN, D), lambda i: (i, 0))]
            + [pl.BlockSpec((2, BN, D), lambda i: (0, i, 0))] * NCHUNK
            + [pl.BlockSpec((NW, BN), lambda i: (0, i))] * NCHUNK
            + [
                pl.BlockSpec((D, D), lambda i: (0, 0)),
                pl.BlockSpec((1, D), lambda i: (0, 0)),
                pl.BlockSpec((D, D), lambda i: (0, 0)),
                pl.BlockSpec((1, D), lambda i: (0, 0)),
                pl.BlockSpec((1, D), lambda i: (0, 0)),
                pl.BlockSpec((1, D), lambda i: (0, 0)),
                pl.BlockSpec((D, 2 * D), lambda i: (0, 0)),
                pl.BlockSpec((1, 2 * D), lambda i: (0, 0)),
                pl.BlockSpec((2 * D, D), lambda i: (0, 0)),
                pl.BlockSpec((1, D), lambda i: (0, 0)),
            ]
        ),
        out_specs=pl.BlockSpec((BN, D), lambda i: (i, 0)),
        out_shape=jax.ShapeDtypeStruct((NP, D), f32),
    )(xp, h, *agg_parts, *deg_parts, W_self, row1(b_self), W_agg, row1(b_agg),
      row1(ln2_g), row1(ln2_b), W_ffn1, row1(b_ffn1), W_ffn2, row1(b_ffn2))

    return (out[:N], edge_repr)


# speedup vs baseline: 1.0587x; 1.0587x over previous
import functools

import jax
import jax.numpy as jnp
from jax import lax
from jax.experimental import pallas as pl
from jax.experimental.pallas import tpu as pltpu
from jax.experimental.pallas import tpu_sc as plsc

NW = 32
SB = 128
SBS = 80
BE = 640
BN = 512
NCHUNK = 4

_INV_SQRT2 = 0.7071067811865476


def _gelu(z):
    return 0.5 * z * (1.0 + lax.erf(z * _INV_SQRT2))


def _ln_rows(xb, g, b, eps=1e-5):
    m = jnp.mean(xb, axis=1, keepdims=True)
    v = jnp.mean((xb - m) ** 2, axis=1, keepdims=True)
    return (xb - m) / jnp.sqrt(v + eps) * g + b


def _pack2(a, b):
    au = lax.bitcast_convert_type(a.astype(jnp.bfloat16).astype(jnp.float32),
                                  jnp.int32)
    bu = lax.bitcast_convert_type(b.astype(jnp.bfloat16).astype(jnp.float32),
                                  jnp.int32)
    return lax.bitcast_convert_type(au | lax.shift_right_logical(bu, 16),
                                    jnp.float32)


def _unpack_hi(p):
    u = lax.bitcast_convert_type(p, jnp.int32)
    return lax.bitcast_convert_type(u & jnp.int32(-65536), jnp.float32)


def _unpack_lo(p):
    u = lax.bitcast_convert_type(p, jnp.int32)
    return lax.bitcast_convert_type(lax.shift_left(u, 16), jnp.float32)


def _prep_body(x_ref, g_ref, b_ref, W_ref, bc_ref, h_ref, ts_ref, td_ref):
    h = _ln_rows(x_ref[...], g_ref[...], b_ref[...])
    h_ref[...] = h
    P = jnp.dot(h, W_ref[...], preferred_element_type=jnp.float32) + bc_ref[...]
    ts_ref[...] = _pack2(P[:, :128], P[:, 128:256])
    td_ref[...] = P[:, 256:]


def _mlp_body(gs_ref, gd_ref, v_ref, emb_ref, Wem_ref, Weg_ref, Wm2_ref,
              bm2_ref, wg2_ref, bg2_ref, id_ref, out_ref):
    gs = gs_ref[...]
    gd = gd_ref[...]
    e = emb_ref[...]
    em = jnp.dot(e, Wem_ref[...], preferred_element_type=jnp.float32)
    eg = jnp.dot(e, Weg_ref[...], preferred_element_type=jnp.float32)
    mpre = _unpack_hi(gs) + em
    gpre = gd + _unpack_lo(gs) + eg
    msg = jnp.dot(_gelu(mpre), Wm2_ref[...],
                  preferred_element_type=jnp.float32) + bm2_ref[...]
    glog = jnp.sum(_gelu(gpre) * wg2_ref[...], axis=1, keepdims=True) + bg2_ref[0, 0]
    mg = msg * jax.nn.sigmoid(glog)
    ident = id_ref[...]
    vb = v_ref[...].reshape(BE // 128, 128)
    for c in range(BE // 128):
        vcol = jnp.sum(ident * vb[c:c + 1, :], axis=1, keepdims=True)
        out_ref[c * 128:(c + 1) * 128, :] = mg[c * 128:(c + 1) * 128, :] * vcol


def kernel(x, edge_src, edge_dst, edge_emb, time_weight, message_node_scale,
           ln1_g, ln1_b, ln2_g, ln2_b, W_self, b_self, W_msg1, b_msg1,
           W_msg2, b_msg2, W_gate1, b_gate1, W_gate2, b_gate2, W_agg, b_agg,
           W_ffn1, b_ffn1, W_ffn2, b_ffn2):
    N, D = x.shape
    E = edge_src.shape[0]
    f32 = jnp.float32

    NP = -(-N // 2048) * 2048
    n_stripe = NP // 16 // SBS

    units = E // 2560
    w = [3, 5, 5, 3][:NCHUNK] if NCHUNK == 4 else [1] * NCHUNK
    tot = sum(w)
    uk = [units * wi // tot for wi in w]
    uk[-1] += units - sum(uk)
    cuts = [0]
    for k in range(NCHUNK):
        cuts.append(cuts[-1] + 2560 * uk[k])

    xp = jnp.pad(x, ((0, NP - N), (0, 0)))
    src_p = edge_src.astype(jnp.int32)
    dst_p = edge_dst.astype(jnp.int32)
    tw_f = time_weight.reshape(E)
    s_f = jnp.pad(message_node_scale, ((0, NP - N), (0, 0))).reshape(NP)

    Wcat = jnp.concatenate([W_msg1[:D], W_gate1[D:2 * D], W_gate1[:D]], axis=1)
    bcat = jnp.concatenate([b_msg1, jnp.zeros((D,), f32), b_gate1])[None, :]
    Wem = W_msg1[D:]
    Weg = W_gate1[2 * D:]
    ident = jnp.eye(128, dtype=f32)

    row1 = lambda a: a[None, :]

    h, t_src, t_dst = pl.pallas_call(
        _prep_body,
        grid=(NP // BN,),
        in_specs=[
            pl.BlockSpec((BN, D), lambda i: (i, 0)),
            pl.BlockSpec((1, D), lambda i: (0, 0)),
            pl.BlockSpec((1, D), lambda i: (0, 0)),
            pl.BlockSpec((D, 3 * D), lambda i: (0, 0)),
            pl.BlockSpec((1, 3 * D), lambda i: (0, 0)),
        ],
        out_specs=[
            pl.BlockSpec((BN, D), lambda i: (i, 0)),
            pl.BlockSpec((BN, D), lambda i: (i, 0)),
            pl.BlockSpec((BN, D), lambda i: (i, 0)),
        ],
        out_shape=[
            jax.ShapeDtypeStruct((NP, D), f32),
            jax.ShapeDtypeStruct((NP, D), f32),
            jax.ShapeDtypeStruct((NP, D), f32),
        ],
    )(xp, row1(ln1_g), row1(ln1_b), Wcat, bcat)

    mesh = plsc.VectorSubcoreMesh(core_axis_name="c", subcore_axis_name="s")
    sc_params = pltpu.CompilerParams(use_tc_tiling_on_sc=False,
                                     needs_layout_passes=False)

    def make_gather(e0, ne):
        b0 = e0 // SB
        nb = ne // SB
        nb_lo = nb // NW
        rem = nb - nb_lo * NW

        @functools.partial(
            pl.kernel,
            out_type=[
                jax.ShapeDtypeStruct((ne, D), f32),
                jax.ShapeDtypeStruct((ne, D), f32),
                jax.ShapeDtypeStruct((nb, SB), f32),
            ],
            mesh=mesh,
            scratch_types=[
                pltpu.VMEM((SB,), jnp.int32),
                pltpu.VMEM((SB,), jnp.int32),
                pltpu.VMEM((SB,), f32),
                pltpu.VMEM((SB,), f32),
                pltpu.VMEM((NP,), f32),
                pltpu.VMEM((SB, D), f32),
                pltpu.VMEM((SB, D), f32),
                pltpu.SemaphoreType.DMA,
                pltpu.SemaphoreType.DMA,
            ],
            compiler_params=sc_params,
        )
        def _gather(ts_hbm, td_hbm, src_hbm, dst_hbm, tw_hbm, s_hbm,
                    gsrc_hbm, gdst_hbm, v_hbm,
                    isrc, idst, twb, vb, s_v, bufA, bufB, semA, semB):
            wid = lax.axis_index("s") * 2 + lax.axis_index("c")
            base_blk = nb_lo * wid + jnp.minimum(wid, rem)
            pltpu.sync_copy(s_hbm, s_v)

            def body(g, carry):
                lblk = jnp.minimum(base_blk + g, nb - 1)
                base = e0 + lblk * SB
                lbase = lblk * SB
                pltpu.sync_copy(src_hbm.at[pl.ds(base, SB)], isrc)
                pltpu.sync_copy(dst_hbm.at[pl.ds(base, SB)], idst)
                pltpu.sync_copy(tw_hbm.at[pl.ds(base, SB)], twb)
                cpA = pltpu.async_copy(ts_hbm.at[isrc], bufA, semA)
                cpB = pltpu.async_copy(td_hbm.at[idst], bufB, semB)
                for i in range(SB // 16):
                    sl = pl.ds(i * 16, 16)
                    ss = plsc.load_gather(s_v, [isrc[sl]])
                    sd = plsc.load_gather(s_v, [idst[sl]])
                    vb[sl] = 0.5 * twb[sl] * (ss + sd)
                cpA.wait()
                cpB.wait()
                pltpu.sync_copy(bufA, gsrc_hbm.at[pl.ds(lbase, SB)])
                pltpu.sync_copy(bufB, gdst_hbm.at[pl.ds(lbase, SB)])
                pltpu.sync_copy(vb, v_hbm.at[lblk])
                return carry

            lax.fori_loop(0, nb_lo + (1 if rem else 0), body, 0)

        return _gather

    def make_scatter(e0, ne):
        n_sc_blk = ne // (NW * SBS)

        @functools.partial(
            pl.kernel,
            out_type=[
                jax.ShapeDtypeStruct((2, NP, D), f32),
                jax.ShapeDtypeStruct((NW, NP), f32),
            ],
            mesh=mesh,
            scratch_types=[
                pltpu.VMEM((SBS, D), f32),
                pltpu.VMEM((SBS,), jnp.int32),
                pltpu.VMEM((NP,), f32),
                pltpu.VMEM_SHARED((NP, D), f32),
            ],
            compiler_params=sc_params,
        )
        def _scatter(er_hbm, dst_hbm, z_hbm, agg_hbm, deg_hbm, rows, dstb,
                     deg_v, agg_sh):
            c = lax.axis_index("c")
            s = lax.axis_index("s")
            wid = s * 2 + c
            per_w = ne // NW
            zero16 = jnp.zeros((16,), f32)
            ones16 = jnp.ones((16,), f32)

            def zdeg(i, carry):
                deg_v[pl.ds(i * 16, 16)] = zero16
                return carry
            lax.fori_loop(0, NP // 16, zdeg, 0)

            pltpu.sync_copy(z_hbm, rows)
            for j in range(n_stripe):
                pltpu.sync_copy(rows,
                                agg_sh.at[pl.ds(s * (NP // 16) + j * SBS, SBS)])
            plsc.subcore_barrier()

            def body(g, carry):
                lbase = wid * per_w + g * SBS
                pltpu.sync_copy(dst_hbm.at[pl.ds(e0 + lbase, SBS)], dstb)
                pltpu.sync_copy(er_hbm.at[pl.ds(lbase, SBS)], rows)
                pltpu.sync_copy(rows, agg_sh.at[dstb], add=True)
                for i in range(SBS // 16):
                    sl = pl.ds(i * 16, 16)
                    plsc.addupdate_scatter(deg_v, [dstb[sl]], ones16)
                return carry

            lax.fori_loop(0, n_sc_blk, body, 0)
            plsc.subcore_barrier()

            for j in range(n_stripe):
                r0 = s * (NP // 16) + j * SBS
                pltpu.sync_copy(agg_sh.at[pl.ds(r0, SBS)], rows)
                pltpu.sync_copy(rows, agg_hbm.at[c].at[pl.ds(r0, SBS)])
            pltpu.sync_copy(deg_v, deg_hbm.at[wid])

        return _scatter

    zrows = jnp.zeros((SBS, D), f32)
    er_chunks, agg_parts, deg_parts = [], [], []
    for k in range(NCHUNK):
        e0, e1 = cuts[k], cuts[k + 1]
        ne = e1 - e0
        g_src, g_dst, v_blk = make_gather(e0, ne)(
            t_src, t_dst, src_p, dst_p, tw_f, s_f)
        v3 = v_blk.reshape(ne // SB, 1, SB)
        eoff = e0 // BE
        DE = edge_emb.shape[1]
        er_k = pl.pallas_call(
            _mlp_body,
            grid=(ne // BE,),
            in_specs=[
                pl.BlockSpec((BE, D), lambda i: (i, 0)),
                pl.BlockSpec((BE, D), lambda i: (i, 0)),
                pl.BlockSpec((BE // SB, 1, SB), lambda i: (i, 0, 0)),
                pl.BlockSpec((BE, DE), lambda i, o=eoff: (i + o, 0)),
                pl.BlockSpec((DE, D), lambda i: (0, 0)),
                pl.BlockSpec((DE, D), lambda i: (0, 0)),
                pl.BlockSpec((D, D), lambda i: (0, 0)),
                pl.BlockSpec((1, D), lambda i: (0, 0)),
                pl.BlockSpec((1, D), lambda i: (0, 0)),
                pl.BlockSpec((1, 1), lambda i: (0, 0)),
                pl.BlockSpec((128, 128), lambda i: (0, 0)),
            ],
            out_specs=pl.BlockSpec((BE, D), lambda i: (i, 0)),
            out_shape=jax.ShapeDtypeStruct((ne, D), f32),
        )(g_src, g_dst, v3, edge_emb, Wem, Weg,
          W_msg2, row1(b_msg2), W_gate2.reshape(1, D), b_gate2.reshape(1, 1),
          ident)
        er_chunks.append(er_k)
        ap_k, dp_k = make_scatter(e0, ne)(er_k, dst_p, zrows)
        agg_parts.append(ap_k)
        deg_parts.append(dp_k)

    edge_repr = (er_chunks[0] if NCHUNK == 1
                 else jnp.concatenate(er_chunks, axis=0))

    def _final_chunked(*refs):
        x_ref, h_ref = refs[0], refs[1]
        aps = refs[2:2 + NCHUNK]
        dps = refs[2 + NCHUNK:2 + 2 * NCHUNK]
        (Ws_ref, bs_ref, Wa_ref, ba_ref, g2_ref, b2_ref,
         Wf1_ref, bf1_ref, Wf2_ref, bf2_ref, out_ref) = refs[2 + 2 * NCHUNK:]
        deg = sum(jnp.sum(d[...], axis=0) for d in dps)
        agg_s = sum(a[0] + a[1] for a in aps)
        agg = agg_s / jnp.clip(deg, 1.0, None)[:, None]
        upd = (jnp.dot(h_ref[...], Ws_ref[...],
                       preferred_element_type=jnp.float32) + bs_ref[...]
               + jnp.dot(agg, Wa_ref[...],
                         preferred_element_type=jnp.float32) + ba_ref[...])
        o1 = x_ref[...] + upd
        f = _ln_rows(o1, g2_ref[...], b2_ref[...])
        ffn = jnp.dot(_gelu(jnp.dot(f, Wf1_ref[...],
                                    preferred_element_type=jnp.float32)
                            + bf1_ref[...]),
                      Wf2_ref[...], preferred_element_type=jnp.float32) + bf2_ref[...]
        out_ref[...] = o1 + ffn

    out = pl.pallas_call(
        _final_chunked,
        grid=(NP // BN,),
        in_specs=(
            [pl.BlockSpec((BN, D), lambda i: (i, 0)),
             pl.BlockSpec((BN, D), lambda i: (i, 0))]
            + [pl.BlockSpec((2, BN, D), lambda i: (0, i, 0))] * NCHUNK
            + [pl.BlockSpec((NW, BN), lambda i: (0, i))] * NCHUNK
            + [
                pl.BlockSpec((D, D), lambda i: (0, 0)),
                pl.BlockSpec((1, D), lambda i: (0, 0)),
                pl.BlockSpec((D, D), lambda i: (0, 0)),
                pl.BlockSpec((1, D), lambda i: (0, 0)),
                pl.BlockSpec((1, D), lambda i: (0, 0)),
                pl.BlockSpec((1, D), lambda i: (0, 0)),
                pl.BlockSpec((D, 2 * D), lambda i: (0, 0)),
                pl.BlockSpec((1, 2 * D), lambda i: (0, 0)),
                pl.BlockSpec((2 * D, D), lambda i: (0, 0)),
                pl.BlockSpec((1, D), lambda i: (0, 0)),
            ]
        ),
        out_specs=pl.BlockSpec((BN, D), lambda i: (i, 0)),
        out_shape=jax.ShapeDtypeStruct((NP, D), f32),
    )(xp, h, *agg_parts, *deg_parts, W_self, row1(b_self), W_agg, row1(b_agg),
      row1(ln2_g), row1(ln2_b), W_ffn1, row1(b_ffn1), W_ffn2, row1(b_ffn2))

    return (out[:N], edge_repr)
